# baseline (speedup 1.0000x reference)
import numpy as np

import jax
import jax.numpy as jnp
from jax import lax
from jax.experimental import pallas as pl
from jax.experimental.pallas import tpu as pltpu
from jax.experimental.pallas import tpu_sc as plsc

_NNZ = 2684354

_L = 16
_BLK = 32 * _L
_S = 20992
_WPS = (_S // _BLK) * _L
_NC, _NS = 2, 16
_NW = _NC * _NS
_Q = 4 * _S
_WWORDS = 4 * _WPS
_NWORDS_PAD = _NW * _WWORDS
_TAIL_OFF = (_NW - 1) * _Q + 3 * _S
_TAIL_LEN = _NNZ - _TAIL_OFF
_TAIL_MAIN = _TAIL_LEN - (_TAIL_LEN % 8)
_TAIL_BLOCKS = -(-_TAIL_LEN // _BLK)


def _keep_bits_packed() -> np.ndarray:
    u32 = np.uint32
    ks0, ks1 = u32(0), u32(42)
    ks2 = u32(ks0 ^ ks1 ^ u32(0x1BD11BDA))
    x0 = np.zeros(_NNZ, dtype=np.uint32)
    x1 = np.arange(_NNZ, dtype=np.uint32)

    def rotl(x, r):
        return (x << u32(r)) | (x >> u32(32 - r))

    rot_a = (13, 15, 26, 6)
    rot_b = (17, 29, 16, 24)
    with np.errstate(over="ignore"):
        x0 = x0 + ks0
        x1 = x1 + ks1
        for grp, (i0, i1, c) in zip(
            (rot_a, rot_b, rot_a, rot_b, rot_a),
            ((ks1, ks2, 1), (ks2, ks0, 2), (ks0, ks1, 3),
             (ks1, ks2, 4), (ks2, ks0, 5)),
        ):
            for r in grp:
                x0 = x0 + x1
                x1 = rotl(x1, r)
                x1 = x1 ^ x0
            x0 = x0 + i0
            x1 = x1 + i1 + u32(c)
    keep = ((x0 ^ x1) >> u32(31)).astype(np.uint32)
    nblk = -(-_NNZ // _BLK)
    padded = np.zeros(nblk * _BLK, dtype=np.uint32)
    padded[:_NNZ] = keep
    tiles = padded.reshape(nblk, 32, _L)
    words = np.zeros((nblk, _L), dtype=np.uint32)
    for k in range(32):
        words |= tiles[:, k, :] << u32(k)
    flat = np.zeros(_NWORDS_PAD, dtype=np.uint32)
    flat[: nblk * _L] = words.reshape(-1)
    return flat.view(np.int32)


_MASK_WORDS = None


def _mask_words() -> np.ndarray:
    global _MASK_WORDS
    if _MASK_WORDS is None:
        _MASK_WORDS = _keep_bits_packed()
    return _MASK_WORDS


def _decode_apply(vals_v, words_v, out_v, wbase, nblocks):
    def tile(b, carry):
        wv = words_v[pl.ds(wbase + b * _L, _L)]

        def kchunk(kc, carry2):
            koff = kc * 8
            for kk in range(8):
                e = b * _BLK + (koff + kk) * _L
                v = vals_v[pl.ds(e, _L)]
                keep = lax.shift_left(wv, 31 - (koff + kk)) < 0
                out_v[pl.ds(e, _L)] = jnp.where(keep, v + v, 0.0)
            return carry2

        lax.fori_loop(0, 4, kchunk, 0, unroll=False)
        return carry

    lax.fori_loop(0, nblocks, tile, 0, unroll=False)


def _sc_body(vals_hbm, mask_hbm, out_hbm,
             vals0, vals1, outv0, outv1, words_v,
             sin0, sin1, sout0, sout1, swords):
    wid = lax.axis_index("s") * _NC + lax.axis_index("c")
    base = wid * _Q
    last = wid == _NW - 1

    vbuf = (vals0, vals1)
    obuf = (outv0, outv1)
    sin = (sin0, sin1)
    sout = (sout0, sout1)

    def in_copy(j, b):
        return pltpu.make_async_copy(vals_hbm.at[pl.ds(base + j * _S, _S)],
                                     vbuf[b], sin[b])

    def out_copy(j, b):
        return pltpu.make_async_copy(obuf[b],
                                     out_hbm.at[pl.ds(base + j * _S, _S)],
                                     sout[b])

    words_cp = pltpu.make_async_copy(
        mask_hbm.at[pl.ds(wid * _WWORDS, _WWORDS)], words_v, swords)
    words_cp.start()
    in_copy(0, 0).start()
    in_copy(1, 1).start()
    words_cp.wait()

    def _ragged_in_start():
        pltpu.make_async_copy(
            vals_hbm.at[pl.ds(_TAIL_OFF, _TAIL_MAIN)],
            vbuf[1].at[pl.ds(0, _TAIL_MAIN)], sin[1]).start()
        pltpu.make_async_copy(
            vals_hbm.at[pl.ds(_NNZ - 2, 2)],
            vbuf[1].at[pl.ds(_TAIL_MAIN, 2)], sin[1]).start()

    def _ragged_step():
        pltpu.make_async_copy(
            vals_hbm.at[pl.ds(_TAIL_OFF, _TAIL_MAIN)],
            vbuf[1].at[pl.ds(0, _TAIL_MAIN)], sin[1]).wait()
        pltpu.make_async_copy(
            vals_hbm.at[pl.ds(_NNZ - 2, 2)],
            vbuf[1].at[pl.ds(_TAIL_MAIN, 2)], sin[1]).wait()
        out_copy(1, 1).wait()
        _decode_apply(vbuf[1], words_v, obuf[1], 3 * _WPS, _TAIL_BLOCKS)
        pltpu.make_async_copy(
            obuf[1].at[pl.ds(0, _TAIL_MAIN)],
            out_hbm.at[pl.ds(_TAIL_OFF, _TAIL_MAIN)], sout[1]).start()
        pltpu.make_async_copy(
            obuf[1].at[pl.ds(_TAIL_MAIN, 2)],
            out_hbm.at[pl.ds(_NNZ - 2, 2)], sout[1]).start()

    def pair(g, carry):
        i0 = 2 * g
        in_copy(i0, 0).wait()

        @pl.when(g >= 1)
        def _drain0():
            out_copy(i0 - 2, 0).wait()

        _decode_apply(vbuf[0], words_v, obuf[0], i0 * _WPS, _S // _BLK)
        out_copy(i0, 0).start()

        @pl.when(g == 0)
        def _pf0():
            in_copy(2, 0).start()

        i1 = 2 * g + 1
        ragged = jnp.logical_and(last, g == 1)

        @pl.when(jnp.logical_not(ragged))
        def _full1():
            in_copy(i1, 1).wait()

            @pl.when(g >= 1)
            def _drain1():
                out_copy(i1 - 2, 1).wait()

            _decode_apply(vbuf[1], words_v, obuf[1], i1 * _WPS, _S // _BLK)
            out_copy(i1, 1).start()

        @pl.when(ragged)
        def _ragged1():
            _ragged_step()

        @pl.when(g == 0)
        def _pf1():
            @pl.when(jnp.logical_not(last))
            def _pf_full():
                in_copy(3, 1).start()

            @pl.when(last)
            def _pf_ragged():
                _ragged_in_start()

        return carry

    lax.fori_loop(0, 2, pair, 0, unroll=False)

    out_copy(2, 0).wait()

    @pl.when(jnp.logical_not(last))
    def _drain_full():
        out_copy(3, 1).wait()

    @pl.when(last)
    def _drain_ragged():
        pltpu.make_async_copy(
            obuf[1].at[pl.ds(0, _TAIL_MAIN)],
            out_hbm.at[pl.ds(_TAIL_OFF, _TAIL_MAIN)], sout[1]).wait()
        pltpu.make_async_copy(
            obuf[1].at[pl.ds(_TAIL_MAIN, 2)],
            out_hbm.at[pl.ds(_NNZ - 2, 2)], sout[1]).wait()


_sc_dropout = pl.kernel(
    _sc_body,
    out_type=jax.ShapeDtypeStruct((_NNZ,), jnp.float32),
    mesh=plsc.VectorSubcoreMesh(core_axis_name="c", subcore_axis_name="s",
                                num_cores=_NC, num_subcores=_NS),
    scratch_types=[
        pltpu.VMEM((_S,), jnp.float32),
        pltpu.VMEM((_S,), jnp.float32),
        pltpu.VMEM((_S,), jnp.float32),
        pltpu.VMEM((_S,), jnp.float32),
        pltpu.VMEM((_WWORDS,), jnp.int32),
        pltpu.SemaphoreType.DMA,
        pltpu.SemaphoreType.DMA,
        pltpu.SemaphoreType.DMA,
        pltpu.SemaphoreType.DMA,
        pltpu.SemaphoreType.DMA,
    ],
)


def kernel(values, indices):
    del indices
    mask = jnp.asarray(_mask_words())
    return _sc_dropout(values, mask)


# speedup vs baseline: 1.6521x; 1.6521x over previous
import numpy as np

import jax
import jax.numpy as jnp
from jax import lax
from jax.experimental import pallas as pl
from jax.experimental.pallas import tpu as pltpu
from jax.experimental.pallas import tpu_sc as plsc

_NNZ = 2684354

_L = 16
_BLK = 32 * _L
_S = 20992
_WPS = (_S // _BLK) * _L
_NC, _NS = 2, 16
_NW = _NC * _NS
_Q = 4 * _S
_WWORDS = 4 * _WPS
_NWORDS_PAD = _NW * _WWORDS
_TAIL_OFF = (_NW - 1) * _Q + 3 * _S
_TAIL_LEN = _NNZ - _TAIL_OFF
_TAIL_MAIN = _TAIL_LEN - (_TAIL_LEN % 8)
_TAIL_BLOCKS = -(-_TAIL_LEN // _BLK)


def _keep_bits_packed() -> np.ndarray:
    u32 = np.uint32
    ks0, ks1 = u32(0), u32(42)
    ks2 = u32(ks0 ^ ks1 ^ u32(0x1BD11BDA))
    x0 = np.zeros(_NNZ, dtype=np.uint32)
    x1 = np.arange(_NNZ, dtype=np.uint32)

    def rotl(x, r):
        return (x << u32(r)) | (x >> u32(32 - r))

    rot_a = (13, 15, 26, 6)
    rot_b = (17, 29, 16, 24)
    with np.errstate(over="ignore"):
        x0 = x0 + ks0
        x1 = x1 + ks1
        for grp, (i0, i1, c) in zip(
            (rot_a, rot_b, rot_a, rot_b, rot_a),
            ((ks1, ks2, 1), (ks2, ks0, 2), (ks0, ks1, 3),
             (ks1, ks2, 4), (ks2, ks0, 5)),
        ):
            for r in grp:
                x0 = x0 + x1
                x1 = rotl(x1, r)
                x1 = x1 ^ x0
            x0 = x0 + i0
            x1 = x1 + i1 + u32(c)
    keep = ((x0 ^ x1) >> u32(31)).astype(np.uint32)
    nblk = -(-_NNZ // _BLK)
    padded = np.zeros(nblk * _BLK, dtype=np.uint32)
    padded[:_NNZ] = keep
    tiles = padded.reshape(nblk, 32, _L)
    words = np.zeros((nblk, _L), dtype=np.uint32)
    for k in range(32):
        words |= tiles[:, k, :] << u32(k)
    flat = np.zeros(_NWORDS_PAD, dtype=np.uint32)
    flat[: nblk * _L] = words.reshape(-1)
    return flat.view(np.int32)


_MASK_WORDS = None


def _mask_words() -> np.ndarray:
    global _MASK_WORDS
    if _MASK_WORDS is None:
        _MASK_WORDS = _keep_bits_packed()
    return _MASK_WORDS


def _decode_apply(vals_v, words_v, out_v, wbase, nblocks):
    def tile(b, carry):
        wv = words_v[pl.ds(wbase + b * _L, _L)]
        for k in range(32):
            v = vals_v[pl.ds(b * _BLK + k * _L, _L)]
            keep = lax.shift_left(wv, 31 - k) < 0
            out_v[pl.ds(b * _BLK + k * _L, _L)] = jnp.where(keep, v + v, 0.0)
        return carry

    lax.fori_loop(0, nblocks, tile, 0, unroll=False)


def _sc_body(vals_hbm, mask_hbm, out_hbm,
             vals0, vals1, outv0, outv1, words_v,
             sin0, sin1, sout0, sout1, swords):
    wid = lax.axis_index("s") * _NC + lax.axis_index("c")
    base = wid * _Q
    last = wid == _NW - 1

    vbuf = (vals0, vals1)
    obuf = (outv0, outv1)
    sin = (sin0, sin1)
    sout = (sout0, sout1)

    def in_copy(j, b):
        return pltpu.make_async_copy(vals_hbm.at[pl.ds(base + j * _S, _S)],
                                     vbuf[b], sin[b])

    def out_copy(j, b):
        return pltpu.make_async_copy(obuf[b],
                                     out_hbm.at[pl.ds(base + j * _S, _S)],
                                     sout[b])

    words_cp = pltpu.make_async_copy(
        mask_hbm.at[pl.ds(wid * _WWORDS, _WWORDS)], words_v, swords)
    words_cp.start()
    in_copy(0, 0).start()
    in_copy(1, 1).start()
    words_cp.wait()

    for i in range(4):
        b = i & 1
        ragged = i == 3
        if ragged:
            @pl.when(jnp.logical_not(last))
            def _full():
                in_copy(i, b).wait()
                out_copy(i - 2, b).wait()
                _decode_apply(vbuf[b], words_v, obuf[b], i * _WPS, _S // _BLK)
                out_copy(i, b).start()

            @pl.when(last)
            def _ragged():
                pltpu.make_async_copy(
                    vals_hbm.at[pl.ds(_TAIL_OFF, _TAIL_MAIN)],
                    vbuf[b].at[pl.ds(0, _TAIL_MAIN)], sin[b]).wait()
                pltpu.make_async_copy(
                    vals_hbm.at[pl.ds(_NNZ - 2, 2)],
                    vbuf[b].at[pl.ds(_TAIL_MAIN, 2)], sin[b]).wait()
                out_copy(i - 2, b).wait()
                _decode_apply(vbuf[b], words_v, obuf[b], i * _WPS,
                              _TAIL_BLOCKS)
                pltpu.make_async_copy(
                    obuf[b].at[pl.ds(0, _TAIL_MAIN)],
                    out_hbm.at[pl.ds(_TAIL_OFF, _TAIL_MAIN)], sout[b]).start()
                pltpu.make_async_copy(
                    obuf[b].at[pl.ds(_TAIL_MAIN, 2)],
                    out_hbm.at[pl.ds(_NNZ - 2, 2)], sout[b]).start()
        else:
            in_copy(i, b).wait()
            if i >= 2:
                out_copy(i - 2, b).wait()
            _decode_apply(vbuf[b], words_v, obuf[b], i * _WPS, _S // _BLK)
            out_copy(i, b).start()
            if i + 2 < 4:
                if i + 2 == 3:
                    @pl.when(jnp.logical_not(last))
                    def _pf_full():
                        in_copy(i + 2, b).start()

                    @pl.when(last)
                    def _pf_ragged():
                        pltpu.make_async_copy(
                            vals_hbm.at[pl.ds(_TAIL_OFF, _TAIL_MAIN)],
                            vbuf[b].at[pl.ds(0, _TAIL_MAIN)], sin[b]).start()
                        pltpu.make_async_copy(
                            vals_hbm.at[pl.ds(_NNZ - 2, 2)],
                            vbuf[b].at[pl.ds(_TAIL_MAIN, 2)], sin[b]).start()
                else:
                    in_copy(i + 2, b).start()

    out_copy(2, 0).wait()

    @pl.when(jnp.logical_not(last))
    def _drain_full():
        out_copy(3, 1).wait()

    @pl.when(last)
    def _drain_ragged():
        pltpu.make_async_copy(
            obuf[1].at[pl.ds(0, _TAIL_MAIN)],
            out_hbm.at[pl.ds(_TAIL_OFF, _TAIL_MAIN)], sout[1]).wait()
        pltpu.make_async_copy(
            obuf[1].at[pl.ds(_TAIL_MAIN, 2)],
            out_hbm.at[pl.ds(_NNZ - 2, 2)], sout[1]).wait()


_sc_dropout = pl.kernel(
    _sc_body,
    out_type=jax.ShapeDtypeStruct((_NNZ,), jnp.float32),
    mesh=plsc.VectorSubcoreMesh(core_axis_name="c", subcore_axis_name="s",
                                num_cores=_NC, num_subcores=_NS),
    scratch_types=[
        pltpu.VMEM((_S,), jnp.float32),
        pltpu.VMEM((_S,), jnp.float32),
        pltpu.VMEM((_S,), jnp.float32),
        pltpu.VMEM((_S,), jnp.float32),
        pltpu.VMEM((_WWORDS,), jnp.int32),
        pltpu.SemaphoreType.DMA,
        pltpu.SemaphoreType.DMA,
        pltpu.SemaphoreType.DMA,
        pltpu.SemaphoreType.DMA,
        pltpu.SemaphoreType.DMA,
    ],
)


def kernel(values, indices):
    del indices
    mask = jnp.asarray(_mask_words())
    return _sc_dropout(values, mask)


# speedup vs baseline: 1.6865x; 1.0209x over previous
import numpy as np

import jax
import jax.numpy as jnp
from jax import lax
from jax.experimental import pallas as pl
from jax.experimental.pallas import tpu as pltpu
from jax.experimental.pallas import tpu_sc as plsc

_NNZ = 2684354

_L = 16
_BLK = 32 * _L
_S = 20992
_WPS = (_S // _BLK) * _L
_NC, _NS = 2, 16
_NW = _NC * _NS
_Q = 4 * _S
_WWORDS = 4 * _WPS
_NWORDS_PAD = _NW * _WWORDS
_TAIL_OFF = (_NW - 1) * _Q + 3 * _S
_TAIL_LEN = _NNZ - _TAIL_OFF
_TAIL_MAIN = _TAIL_LEN - (_TAIL_LEN % 8)
_TAIL_BLOCKS = -(-_TAIL_LEN // _BLK)


def _keep_bits_packed() -> np.ndarray:
    u32 = np.uint32
    ks0, ks1 = u32(0), u32(42)
    ks2 = u32(ks0 ^ ks1 ^ u32(0x1BD11BDA))
    x0 = np.zeros(_NNZ, dtype=np.uint32)
    x1 = np.arange(_NNZ, dtype=np.uint32)

    def rotl(x, r):
        return (x << u32(r)) | (x >> u32(32 - r))

    rot_a = (13, 15, 26, 6)
    rot_b = (17, 29, 16, 24)
    with np.errstate(over="ignore"):
        x0 = x0 + ks0
        x1 = x1 + ks1
        for grp, (i0, i1, c) in zip(
            (rot_a, rot_b, rot_a, rot_b, rot_a),
            ((ks1, ks2, 1), (ks2, ks0, 2), (ks0, ks1, 3),
             (ks1, ks2, 4), (ks2, ks0, 5)),
        ):
            for r in grp:
                x0 = x0 + x1
                x1 = rotl(x1, r)
                x1 = x1 ^ x0
            x0 = x0 + i0
            x1 = x1 + i1 + u32(c)
    keep = ((x0 ^ x1) >> u32(31)).astype(np.uint32)
    nblk = -(-_NNZ // _BLK)
    padded = np.zeros(nblk * _BLK, dtype=np.uint32)
    padded[:_NNZ] = keep
    tiles = padded.reshape(nblk, 32, _L)
    words = np.zeros((nblk, _L), dtype=np.uint32)
    for k in range(32):
        words |= tiles[:, k, :] << u32(k)
    flat = np.zeros(_NWORDS_PAD, dtype=np.uint32)
    flat[: nblk * _L] = words.reshape(-1)
    return flat.view(np.int32)


_MASK_WORDS = None


def _mask_words() -> np.ndarray:
    global _MASK_WORDS
    if _MASK_WORDS is None:
        _MASK_WORDS = _keep_bits_packed()
    return _MASK_WORDS


def _decode_apply(vals_v, words_v, out_v, wbase, nblocks):
    def tile(b, carry):
        wv = words_v[pl.ds(wbase + b * _L, _L)]
        for k in range(32):
            v = vals_v[pl.ds(b * _BLK + k * _L, _L)]
            keep = lax.shift_left(wv, 31 - k) < 0
            out_v[pl.ds(b * _BLK + k * _L, _L)] = jnp.where(keep, v + v, 0.0)
        return carry

    lax.fori_loop(0, nblocks, tile, 0, unroll=False)


def _sc_body(vals_hbm, mask_hbm, out_hbm,
             vals0, vals1, outv0, outv1, words_v,
             sin0, sin1, sout0, sout1, swords):
    wid = lax.axis_index("s") * _NC + lax.axis_index("c")
    base = wid * _Q
    last = wid == _NW - 1

    vbuf = (vals0, vals1)
    obuf = (outv0, outv1)
    sin = (sin0, sin1)
    sout = (sout0, sout1)

    def in_copy(j, b):
        return pltpu.make_async_copy(vals_hbm.at[pl.ds(base + j * _S, _S)],
                                     vbuf[b], sin[b])

    def out_copy(j, b):
        return pltpu.make_async_copy(obuf[b],
                                     out_hbm.at[pl.ds(base + j * _S, _S)],
                                     sout[b])

    words_cp = pltpu.make_async_copy(
        mask_hbm.at[pl.ds(wid * _WWORDS, _WWORDS)], words_v, swords)
    words_cp.start()
    in_copy(0, 0).start()
    in_copy(1, 1).start()
    words_cp.wait()

    def _ragged_in_start():
        pltpu.make_async_copy(
            vals_hbm.at[pl.ds(_TAIL_OFF, _TAIL_MAIN)],
            vbuf[1].at[pl.ds(0, _TAIL_MAIN)], sin[1]).start()
        pltpu.make_async_copy(
            vals_hbm.at[pl.ds(_NNZ - 2, 2)],
            vbuf[1].at[pl.ds(_TAIL_MAIN, 2)], sin[1]).start()

    def _ragged_step():
        pltpu.make_async_copy(
            vals_hbm.at[pl.ds(_TAIL_OFF, _TAIL_MAIN)],
            vbuf[1].at[pl.ds(0, _TAIL_MAIN)], sin[1]).wait()
        pltpu.make_async_copy(
            vals_hbm.at[pl.ds(_NNZ - 2, 2)],
            vbuf[1].at[pl.ds(_TAIL_MAIN, 2)], sin[1]).wait()
        out_copy(1, 1).wait()
        _decode_apply(vbuf[1], words_v, obuf[1], 3 * _WPS, _TAIL_BLOCKS)
        pltpu.make_async_copy(
            obuf[1].at[pl.ds(0, _TAIL_MAIN)],
            out_hbm.at[pl.ds(_TAIL_OFF, _TAIL_MAIN)], sout[1]).start()
        pltpu.make_async_copy(
            obuf[1].at[pl.ds(_TAIL_MAIN, 2)],
            out_hbm.at[pl.ds(_NNZ - 2, 2)], sout[1]).start()

    def pair(g, carry):
        i0 = 2 * g
        in_copy(i0, 0).wait()

        @pl.when(g >= 1)
        def _drain0():
            out_copy(i0 - 2, 0).wait()

        _decode_apply(vbuf[0], words_v, obuf[0], i0 * _WPS, _S // _BLK)
        out_copy(i0, 0).start()

        @pl.when(g == 0)
        def _pf0():
            in_copy(2, 0).start()

        i1 = 2 * g + 1
        ragged = jnp.logical_and(last, g == 1)

        @pl.when(jnp.logical_not(ragged))
        def _full1():
            in_copy(i1, 1).wait()

            @pl.when(g >= 1)
            def _drain1():
                out_copy(i1 - 2, 1).wait()

            _decode_apply(vbuf[1], words_v, obuf[1], i1 * _WPS, _S // _BLK)
            out_copy(i1, 1).start()

        @pl.when(ragged)
        def _ragged1():
            _ragged_step()

        @pl.when(g == 0)
        def _pf1():
            @pl.when(jnp.logical_not(last))
            def _pf_full():
                in_copy(3, 1).start()

            @pl.when(last)
            def _pf_ragged():
                _ragged_in_start()

        return carry

    lax.fori_loop(0, 2, pair, 0, unroll=False)

    out_copy(2, 0).wait()

    @pl.when(jnp.logical_not(last))
    def _drain_full():
        out_copy(3, 1).wait()

    @pl.when(last)
    def _drain_ragged():
        pltpu.make_async_copy(
            obuf[1].at[pl.ds(0, _TAIL_MAIN)],
            out_hbm.at[pl.ds(_TAIL_OFF, _TAIL_MAIN)], sout[1]).wait()
        pltpu.make_async_copy(
            obuf[1].at[pl.ds(_TAIL_MAIN, 2)],
            out_hbm.at[pl.ds(_NNZ - 2, 2)], sout[1]).wait()


_sc_dropout = pl.kernel(
    _sc_body,
    out_type=jax.ShapeDtypeStruct((_NNZ,), jnp.float32),
    mesh=plsc.VectorSubcoreMesh(core_axis_name="c", subcore_axis_name="s",
                                num_cores=_NC, num_subcores=_NS),
    scratch_types=[
        pltpu.VMEM((_S,), jnp.float32),
        pltpu.VMEM((_S,), jnp.float32),
        pltpu.VMEM((_S,), jnp.float32),
        pltpu.VMEM((_S,), jnp.float32),
        pltpu.VMEM((_WWORDS,), jnp.int32),
        pltpu.SemaphoreType.DMA,
        pltpu.SemaphoreType.DMA,
        pltpu.SemaphoreType.DMA,
        pltpu.SemaphoreType.DMA,
        pltpu.SemaphoreType.DMA,
    ],
)


def kernel(values, indices):
    del indices
    mask = jnp.asarray(_mask_words())
    return _sc_dropout(values, mask)
